# baseline (speedup 1.0000x reference)
import functools

import jax
import jax.numpy as jnp
from jax import lax
from jax.experimental import pallas as pl
from jax.experimental.pallas import tpu as pltpu
from jax.experimental.pallas import tpu_sc as plsc

_K = 5
_L = 16
_NC = 2
_NS = 16
_NW = _NC * _NS
_B = 1024
_N = 100000
_BSC = 256
_BTC = _B - _BSC
_RW = _BSC // _NW
_G = 8
_NG = _RW // _G
_P = 6144
_NP = 16
_PC = _P // _L
_PT = 1664
_PTC = _PT // _L
_PE = 32
_PEC = _PE // _L
_BCOL = 768
_BC = _BCOL // _L
_NB = _P // _BCOL
_NBM = _NP * _NB
_RB = 2
_RBD = _RB


def _insert(carry, v):
    t1, t2, t3, t4, t5 = carry
    m1 = jnp.maximum(t1, v)
    c1 = jnp.minimum(t1, v)
    m2 = jnp.maximum(t2, c1)
    c2 = jnp.minimum(t2, c1)
    m3 = jnp.maximum(t3, c2)
    c3 = jnp.minimum(t3, c2)
    m4 = jnp.maximum(t4, c3)
    c4 = jnp.minimum(t4, c3)
    m5 = jnp.maximum(t5, c4)
    return (m1, m2, m3, m4, m5)


def _body(x_hbm, y_hbm, out_hbm, buf0, buf1, tb, eb, rb, y_v, bm_v, st_v,
          out_v, sem0, sem1, sem2, sem3, sem4):
    wid = lax.axis_index("s") * _NC + lax.axis_index("c")
    base = wid * _RW

    pltpu.sync_copy(y_hbm.at[pl.ds(base, _RW)], y_v)

    lanes = lax.iota(jnp.int32, _L)
    lane0 = lanes == 0
    neg_inf = jnp.float32(-jnp.inf)
    ninf_vec = jnp.full((_L,), neg_inf)

    def dma_panel(gg, p, buf, sem):
        return pltpu.async_copy(
            x_hbm.at[pl.ds(gg * _G, _G), pl.ds(p * _P, _P)], buf, sem)

    def wait_panel(gg, p, buf, sem):
        pltpu.make_async_copy(
            x_hbm.at[pl.ds(gg * _G, _G), pl.ds(p * _P, _P)], buf,
            sem).wait()

    def load_stack(r):
        sb = r * _K * _L
        return tuple(st_v[pl.ds(sb + k * _L, _L)] for k in range(_K))

    def store_stack(r, stack):
        sb = r * _K * _L
        for k in range(_K):
            st_v[pl.ds(sb + k * _L, _L)] = stack[k]

    def fix_chunk(buf2d, r, col_off, y_i):
        off = y_i - col_off
        c_y = off // _L
        l_y = off % _L
        vy = buf2d[r, pl.ds(c_y * _L, _L)]
        eq = lanes == l_y
        s = jnp.sum(jnp.where(eq, vy, 0.0))
        buf2d[r, pl.ds(c_y * _L, _L)] = jnp.where(eq, 0.0, vy)
        return s

    def maybe_fix(buf2d, r, col_off, cols, y_i, sy_vec):
        in_span = (y_i >= col_off) & (y_i < col_off + cols)
        s_y = lax.cond(in_span, lambda: fix_chunk(buf2d, r, col_off, y_i),
                       lambda: jnp.float32(0.0))
        return sy_vec + jnp.where(lanes == r, s_y, 0.0)

    gather_dnums = lax.GatherDimensionNumbers(
        offset_dims=(), collapsed_slice_dims=(0,), start_index_map=(0,))

    def cross_lane_max(v):
        for s in (1, 2, 4, 8):
            perm = lax.gather(
                v, (lanes ^ s)[:, None], gather_dnums, slice_sizes=(1,),
                mode=lax.GatherScatterMode.PROMISE_IN_BOUNDS)
            v = jnp.maximum(v, perm)
        return v

    def panel_pass_a(buf, r, col_off, y_i, sy_vec, pack, pack_lane0):
        sy_vec = maybe_fix(buf, r, col_off, _P, y_i, sy_vec)

        def block_body(b, pack):
            cb = b * _BC
            vs = [buf[r, pl.ds((cb + c) * _L, _L)] for c in range(_BC)]
            while len(vs) > 1:
                vs = [jnp.maximum(vs[i], vs[i + 1])
                      for i in range(0, len(vs) - 1, 2)] + (
                          [vs[-1]] if len(vs) % 2 else [])
            m = cross_lane_max(vs[0])
            return jnp.where(lanes == pack_lane0 + b, m, pack)

        pack = lax.fori_loop(0, _NB, block_body, pack)
        return sy_vec, pack

    def direct_span(buf, r, col_off, cols, n_chunks, y_i, sy_vec):
        sy_vec = maybe_fix(buf, r, col_off, cols, y_i, sy_vec)
        stack = load_stack(r)

        def chunk_body(c, carry):
            return _insert(carry, buf[r, pl.ds(c * _L, _L)])

        stack = lax.fori_loop(0, n_chunks, chunk_body, stack, unroll=4)
        store_stack(r, stack)
        return sy_vec

    def group_loop(g, carry):
        loss_acc = carry
        gg = wid * _NG + g

        pltpu.async_copy(
            x_hbm.at[pl.ds(gg * _G, _G), pl.ds(_NP * _P, _PT)], tb, sem2)
        pltpu.async_copy(
            x_hbm.at[pl.ds(gg * _G, _G), pl.ds(_NP * _P + _PT, _PE)], eb,
            sem3)

        for r in range(_G):
            for k in range(_K):
                st_v[pl.ds((r * _K + k) * _L, _L)] = ninf_vec

        ys = []
        for r in range(_G):
            j = g * _G + r
            y_vec = y_v[pl.ds((j // _L) * _L, _L)]
            ys.append(jnp.sum(jnp.where(lanes == (j % _L), y_vec, 0)))

        sy_vec = jnp.zeros((_L,), jnp.float32)

        def pair_body(p2, carry):
            sy_vec = carry
            p0 = p2 * 2
            dma_panel(gg, p0 + 1, buf1, sem1)
            wait_panel(gg, p0, buf0, sem0)
            packs = [ninf_vec] * _G
            for r in range(_G):
                sy_vec, packs[r] = panel_pass_a(buf0, r, p0 * _P, ys[r],
                                                sy_vec, packs[r], 0)

            @pl.when(p0 + 2 < _NP)
            def _prefetch():
                dma_panel(gg, p0 + 2, buf0, sem0)

            wait_panel(gg, p0 + 1, buf1, sem1)
            for r in range(_G):
                sy_vec, packs[r] = panel_pass_a(buf1, r, (p0 + 1) * _P,
                                                ys[r], sy_vec, packs[r], _NB)
            for r in range(_G):
                bm_v[pl.ds((r * (_NP // 2) + p2) * _L, _L)] = packs[r]
            return sy_vec

        sy_vec = lax.fori_loop(0, _NP // 2, pair_body, sy_vec)

        @pl.when(g + 1 < _NG)
        def _prefetch_next_group():
            dma_panel(gg + 1, 0, buf0, sem0)

        pltpu.make_async_copy(
            x_hbm.at[pl.ds(gg * _G, _G), pl.ds(_NP * _P, _PT)], tb,
            sem2).wait()
        pltpu.make_async_copy(
            x_hbm.at[pl.ds(gg * _G, _G), pl.ds(_NP * _P + _PT, _PE)], eb,
            sem3).wait()

        def select_blocks(r):
            iz = jnp.zeros((_L,), jnp.int32)

            def sel_body(kch, carry):
                tb1, tb2, tb3, tb4, tb5, ib1, ib2, ib3, ib4, ib5 = carry
                bm = bm_v[pl.ds((r * (_NP // 2) + kch) * _L, _L)]
                ib = lanes + kch * _L
                s1 = bm > tb1
                n1, i1 = jnp.where(s1, bm, tb1), jnp.where(s1, ib, ib1)
                bm, ib = jnp.where(s1, tb1, bm), jnp.where(s1, ib1, ib)
                s2 = bm > tb2
                n2, i2 = jnp.where(s2, bm, tb2), jnp.where(s2, ib, ib2)
                bm, ib = jnp.where(s2, tb2, bm), jnp.where(s2, ib2, ib)
                s3 = bm > tb3
                n3, i3 = jnp.where(s3, bm, tb3), jnp.where(s3, ib, ib3)
                bm, ib = jnp.where(s3, tb3, bm), jnp.where(s3, ib3, ib)
                s4 = bm > tb4
                n4, i4 = jnp.where(s4, bm, tb4), jnp.where(s4, ib, ib4)
                bm, ib = jnp.where(s4, tb4, bm), jnp.where(s4, ib4, ib)
                s5 = bm > tb5
                n5, i5 = jnp.where(s5, bm, tb5), jnp.where(s5, ib, ib5)
                return (n1, n2, n3, n4, n5, i1, i2, i3, i4, i5)

            (tb1, tb2, tb3, tb4, tb5,
             ib1, ib2, ib3, ib4, ib5) = lax.fori_loop(
                 0, _NP // 2, sel_body,
                 (ninf_vec,) * _K + (iz,) * _K)
            slots = []
            for _ in range(_K):
                m = jnp.max(tb1)
                ffs = plsc.all_reduce_ffs(tb1 == m)
                sel = lanes == ffs
                slots.append(jnp.sum(jnp.where(sel, ib1, 0)))
                tb1 = jnp.where(sel, tb2, tb1)
                tb2 = jnp.where(sel, tb3, tb2)
                tb3 = jnp.where(sel, tb4, tb3)
                tb4 = jnp.where(sel, tb5, tb4)
                tb5 = jnp.where(sel, neg_inf, tb5)
                ib1 = jnp.where(sel, ib2, ib1)
                ib2 = jnp.where(sel, ib3, ib2)
                ib3 = jnp.where(sel, ib4, ib3)
                ib4 = jnp.where(sel, ib5, ib4)
            return slots

        def slot_col(slot):
            pair = slot // _L
            lane = slot % _L
            pnl = 2 * pair + lane // _NB
            blk = lane % _NB
            return pnl * _P + blk * _BCOL

        def fetch_row_blocks(r, cols):
            for k in range(_K):
                pltpu.async_copy(
                    x_hbm.at[gg * _G + r, pl.ds(cols[k], _BCOL)],
                    rb.at[pl.ds(((r % _RBD) * _K + k) * _BCOL, _BCOL)],
                    sem4)

        def pass_b_row(r, cols, y_i):
            stack = load_stack(r)
            for k in range(_K):
                rbo = ((r % _RBD) * _K + k) * _BCOL
                def refix(rbo=rbo, col=cols[k]):
                    off = y_i - col
                    c_y = off // _L
                    l_y = off % _L
                    vy = rb[pl.ds(rbo + c_y * _L, _L)]
                    rb[pl.ds(rbo + c_y * _L, _L)] = jnp.where(
                        lanes == l_y, 0.0, vy)

                in_blk = (y_i >= cols[k]) & (y_i < cols[k] + _BCOL)
                lax.cond(in_blk, refix, lambda: None)

                def chunk_body(c, carry, rbo=rbo):
                    return _insert(carry, rb[pl.ds(rbo + c * _L, _L)])

                stack = lax.fori_loop(0, _BC, chunk_body, stack, unroll=4)
            store_stack(r, stack)

        row_cols = []
        for r in range(_G):
            slots = select_blocks(r)
            row_cols.append([slot_col(s) for s in slots])

        fetch_row_blocks(0, row_cols[0])
        fetch_row_blocks(1, row_cols[1])
        for r in range(_G):
            sy_vec = direct_span(tb, r, _NP * _P, _PT, _PTC, ys[r], sy_vec)
            sy_vec = direct_span(eb, r, _NP * _P + _PT, _PE, _PEC, ys[r],
                                 sy_vec)

        for ph in range(_G // _RB):
            r0 = ph * _RB
            for r in (r0, r0 + 1):
                for k in range(_K):
                    pltpu.make_async_copy(
                        x_hbm.at[gg * _G + r,
                                 pl.ds(row_cols[r][k], _BCOL)],
                        rb.at[pl.ds(((r % _RBD) * _K + k) * _BCOL,
                                    _BCOL)],
                        sem4).wait()
            for r in (r0, r0 + 1):
                pass_b_row(r, row_cols[r], ys[r])
            if ph + 1 < _G // _RB:
                fetch_row_blocks(r0 + 2, row_cols[r0 + 2])
                fetch_row_blocks(r0 + 3, row_cols[r0 + 3])

        for r in range(_G):
            t1, t2, t3, t4, t5 = load_stack(r)
            for _ in range(_K - 1):
                m = jnp.max(t1)
                ffs = plsc.all_reduce_ffs(t1 == m)
                sel = lanes == ffs
                t1 = jnp.where(sel, t2, t1)
                t2 = jnp.where(sel, t3, t2)
                t3 = jnp.where(sel, t4, t3)
                t4 = jnp.where(sel, t5, t4)
                t5 = jnp.where(sel, neg_inf, t5)
            s_topk = jnp.max(t1)
            s_y = jnp.sum(jnp.where(lanes == r, sy_vec, 0.0))
            hinge = jnp.maximum(1.0 + s_topk - s_y, 0.0)
            loss_acc = loss_acc + jnp.where(lane0, hinge, 0.0)
        return loss_acc

    dma_panel(wid * _NG, 0, buf0, sem0)
    loss_acc = lax.fori_loop(0, _NG, group_loop,
                             jnp.zeros((_L,), jnp.float32))

    out_v[...] = loss_acc
    pltpu.sync_copy(out_v, out_hbm.at[wid])


_TCG = _BTC // _G
_TCP = 6400
_TCNP = 16
_TCPC = _TCP // 128
_NST = 5


def _tc_body(y_ref, x_ref, out_ref, t_ref, sy_ref):
    j = pl.program_id(1)
    ycol = y_ref[0]
    idx = lax.broadcasted_iota(jnp.int32, (1, 128), 1)
    neg_inf = jnp.float32(-jnp.inf)
    ninf = jnp.full((_G, 128), neg_inf)

    @pl.when(j == 0)
    def _init():
        for k in range(_NST * _K):
            t_ref[k] = ninf
        sy_ref[...] = jnp.zeros((_G, 128), jnp.float32)

    def insert(stk, v):
        t1, t2, t3, t4, t5 = stk
        m1 = jnp.maximum(t1, v)
        c1 = jnp.minimum(t1, v)
        m2 = jnp.maximum(t2, c1)
        c2 = jnp.minimum(t2, c1)
        m3 = jnp.maximum(t3, c2)
        c3 = jnp.minimum(t3, c2)
        m4 = jnp.maximum(t4, c3)
        c4 = jnp.minimum(t4, c3)
        m5 = jnp.maximum(t5, c4)
        return [m1, m2, m3, m4, m5]

    stks = [[t_ref[ph * _K + k] for k in range(_K)]
            for ph in range(_NST)]
    syv = sy_ref[...]
    for c in range(_TCPC):
        v = x_ref[:, c * 128:(c + 1) * 128]
        cols = idx + (j * _TCP + c * 128)
        v = jnp.where(cols < _N, v, neg_inf)
        eq = cols == ycol
        syv = syv + jnp.where(eq, v, 0.0)
        v = jnp.where(eq, 0.0, v)
        stks[c % _NST] = insert(stks[c % _NST], v)
    for ph in range(_NST):
        for k in range(_K):
            t_ref[ph * _K + k] = stks[ph][k]
    sy_ref[...] = syv

    @pl.when(j == _TCNP - 1)
    def _finish():
        t1, t2, t3, t4, t5 = (t_ref[k] for k in range(_K))
        for ph in range(1, _NST):
            for k in range(_K):
                t1, t2, t3, t4, t5 = insert(
                    [t1, t2, t3, t4, t5], t_ref[ph * _K + k])
        sy = jnp.sum(sy_ref[...], axis=1, keepdims=True)
        for _ in range(_K - 1):
            m = jnp.max(t1, axis=1, keepdims=True)
            eq = t1 == m
            jstar = jnp.min(jnp.where(eq, idx, 10000), axis=1,
                            keepdims=True)
            sel = idx == jstar
            t1 = jnp.where(sel, t2, t1)
            t2 = jnp.where(sel, t3, t2)
            t3 = jnp.where(sel, t4, t3)
            t4 = jnp.where(sel, t5, t4)
            t5 = jnp.where(sel, neg_inf, t5)
        s_topk = jnp.max(t1, axis=1, keepdims=True)
        hinge = jnp.maximum(1.0 + s_topk - sy, 0.0)
        out_ref[...] = jnp.broadcast_to(hinge, (_G, 128))


@jax.jit
def kernel(x, y):
    mesh = plsc.VectorSubcoreMesh(core_axis_name="c", subcore_axis_name="s")
    partials = pl.kernel(
        _body,
        out_type=jax.ShapeDtypeStruct((_NW, _L), jnp.float32),
        mesh=mesh,
        compiler_params=pltpu.CompilerParams(needs_layout_passes=False),
        scratch_types=[
            pltpu.VMEM((_G, _P), jnp.float32),
            pltpu.VMEM((_G, _P), jnp.float32),
            pltpu.VMEM((_G, _PT), jnp.float32),
            pltpu.VMEM((_G, _PE), jnp.float32),
            pltpu.VMEM((_RBD * _K * _BCOL,), jnp.float32),
            pltpu.VMEM((_RW,), jnp.int32),
            pltpu.VMEM((_G * (_NP // 2) * _L,), jnp.float32),
            pltpu.VMEM((_G * _K * _L,), jnp.float32),
            pltpu.VMEM((_L,), jnp.float32),
            pltpu.SemaphoreType.DMA,
            pltpu.SemaphoreType.DMA,
            pltpu.SemaphoreType.DMA,
            pltpu.SemaphoreType.DMA,
            pltpu.SemaphoreType.DMA,
        ],
    )(x, y)

    tc_hinge = pl.pallas_call(
        _tc_body,
        grid=(_TCG, _TCNP),
        in_specs=[
            pl.BlockSpec((1, _G, 1), lambda b, j: (b, 0, 0)),
            pl.BlockSpec((_G, _TCP), lambda b, j: (b + _BSC // _G, j)),
        ],
        out_specs=pl.BlockSpec((_G, 128), lambda b, j: (b, 0)),
        out_shape=jax.ShapeDtypeStruct((_BTC, 128), jnp.float32),
        scratch_shapes=[
            pltpu.VMEM((_NST * _K, _G, 128), jnp.float32),
            pltpu.VMEM((_G, 128), jnp.float32),
        ],
        compiler_params=pltpu.CompilerParams(
            dimension_semantics=("parallel", "arbitrary")),
    )(y[_BSC:].reshape(_TCG, _G, 1), x)

    return ((jnp.sum(partials[:, 0]) + jnp.sum(tc_hinge[:, 0]))
            / jnp.float32(_B))


# speedup vs baseline: 2.2847x; 2.2847x over previous
import functools

import jax
import jax.numpy as jnp
from jax import lax
from jax.experimental import pallas as pl
from jax.experimental.pallas import tpu as pltpu
from jax.experimental.pallas import tpu_sc as plsc

_K = 5
_L = 16
_NC = 2
_NS = 16
_NW = _NC * _NS
_B = 1024
_N = 100000
_RW = _B // _NW
_G = 8
_NG = _RW // _G
_P = 6144
_NP = 16
_PC = _P // _L
_PT = 1664
_PTC = _PT // _L
_PE = 32
_PEC = _PE // _L
_BCOL = 768
_BC = _BCOL // _L
_NB = _P // _BCOL
_NBM = _NP * _NB
_RB = 2
_RBD = _RB


def _insert(carry, v):
    t1, t2, t3, t4, t5 = carry
    m1 = jnp.maximum(t1, v)
    c1 = jnp.minimum(t1, v)
    m2 = jnp.maximum(t2, c1)
    c2 = jnp.minimum(t2, c1)
    m3 = jnp.maximum(t3, c2)
    c3 = jnp.minimum(t3, c2)
    m4 = jnp.maximum(t4, c3)
    c4 = jnp.minimum(t4, c3)
    m5 = jnp.maximum(t5, c4)
    return (m1, m2, m3, m4, m5)


def _body(x_hbm, y_hbm, out_hbm, buf0, buf1, tb, eb, rb, y_v, bm_v, st_v,
          out_v, sem0, sem1, sem2, sem3, sem4):
    wid = lax.axis_index("s") * _NC + lax.axis_index("c")
    base = wid * _RW

    pltpu.sync_copy(y_hbm.at[pl.ds(base, _RW)], y_v)

    lanes = lax.iota(jnp.int32, _L)
    lane0 = lanes == 0
    neg_inf = jnp.float32(-jnp.inf)
    ninf_vec = jnp.full((_L,), neg_inf)

    def dma_panel(gg, p, buf, sem):
        return pltpu.async_copy(
            x_hbm.at[pl.ds(gg * _G, _G), pl.ds(p * _P, _P)], buf, sem)

    def wait_panel(gg, p, buf, sem):
        pltpu.make_async_copy(
            x_hbm.at[pl.ds(gg * _G, _G), pl.ds(p * _P, _P)], buf,
            sem).wait()

    def load_stack(r):
        sb = r * _K * _L
        return tuple(st_v[pl.ds(sb + k * _L, _L)] for k in range(_K))

    def store_stack(r, stack):
        sb = r * _K * _L
        for k in range(_K):
            st_v[pl.ds(sb + k * _L, _L)] = stack[k]

    def fix_chunk(buf2d, r, col_off, y_i):
        off = y_i - col_off
        c_y = off // _L
        l_y = off % _L
        vy = buf2d[r, pl.ds(c_y * _L, _L)]
        eq = lanes == l_y
        s = jnp.sum(jnp.where(eq, vy, 0.0))
        buf2d[r, pl.ds(c_y * _L, _L)] = jnp.where(eq, 0.0, vy)
        return s

    def maybe_fix(buf2d, r, col_off, cols, y_i, sy_vec):
        in_span = (y_i >= col_off) & (y_i < col_off + cols)
        s_y = lax.cond(in_span, lambda: fix_chunk(buf2d, r, col_off, y_i),
                       lambda: jnp.float32(0.0))
        return sy_vec + jnp.where(lanes == r, s_y, 0.0)

    gather_dnums = lax.GatherDimensionNumbers(
        offset_dims=(), collapsed_slice_dims=(0,), start_index_map=(0,))

    def cross_lane_max(v):
        for s in (1, 2, 4, 8):
            perm = lax.gather(
                v, (lanes ^ s)[:, None], gather_dnums, slice_sizes=(1,),
                mode=lax.GatherScatterMode.PROMISE_IN_BOUNDS)
            v = jnp.maximum(v, perm)
        return v

    def panel_pass_a(buf, r, col_off, y_i, sy_vec, pack, pack_lane0):
        sy_vec = maybe_fix(buf, r, col_off, _P, y_i, sy_vec)

        def block_body(b, pack):
            cb = b * _BC
            vs = [buf[r, pl.ds((cb + c) * _L, _L)] for c in range(_BC)]
            while len(vs) > 1:
                vs = [jnp.maximum(vs[i], vs[i + 1])
                      for i in range(0, len(vs) - 1, 2)] + (
                          [vs[-1]] if len(vs) % 2 else [])
            m = cross_lane_max(vs[0])
            return jnp.where(lanes == pack_lane0 + b, m, pack)

        pack = lax.fori_loop(0, _NB, block_body, pack)
        return sy_vec, pack

    def direct_span(buf, r, col_off, cols, n_chunks, y_i, sy_vec):
        sy_vec = maybe_fix(buf, r, col_off, cols, y_i, sy_vec)
        stack = load_stack(r)

        def chunk_body(c, carry):
            return _insert(carry, buf[r, pl.ds(c * _L, _L)])

        stack = lax.fori_loop(0, n_chunks, chunk_body, stack, unroll=4)
        store_stack(r, stack)
        return sy_vec

    def group_loop(g, carry):
        loss_acc = carry
        gg = wid * _NG + g

        pltpu.async_copy(
            x_hbm.at[pl.ds(gg * _G, _G), pl.ds(_NP * _P, _PT)], tb, sem2)
        pltpu.async_copy(
            x_hbm.at[pl.ds(gg * _G, _G), pl.ds(_NP * _P + _PT, _PE)], eb,
            sem3)

        for r in range(_G):
            for k in range(_K):
                st_v[pl.ds((r * _K + k) * _L, _L)] = ninf_vec

        ys = []
        for r in range(_G):
            j = g * _G + r
            y_vec = y_v[pl.ds((j // _L) * _L, _L)]
            ys.append(jnp.sum(jnp.where(lanes == (j % _L), y_vec, 0)))

        sy_vec = jnp.zeros((_L,), jnp.float32)

        def pair_body(p2, carry):
            sy_vec = carry
            p0 = p2 * 2
            dma_panel(gg, p0 + 1, buf1, sem1)
            wait_panel(gg, p0, buf0, sem0)
            packs = [ninf_vec] * _G
            for r in range(_G):
                sy_vec, packs[r] = panel_pass_a(buf0, r, p0 * _P, ys[r],
                                                sy_vec, packs[r], 0)

            @pl.when(p0 + 2 < _NP)
            def _prefetch():
                dma_panel(gg, p0 + 2, buf0, sem0)

            wait_panel(gg, p0 + 1, buf1, sem1)
            for r in range(_G):
                sy_vec, packs[r] = panel_pass_a(buf1, r, (p0 + 1) * _P,
                                                ys[r], sy_vec, packs[r], _NB)
            for r in range(_G):
                bm_v[pl.ds((r * (_NP // 2) + p2) * _L, _L)] = packs[r]
            return sy_vec

        sy_vec = lax.fori_loop(0, _NP // 2, pair_body, sy_vec)

        @pl.when(g + 1 < _NG)
        def _prefetch_next_group():
            dma_panel(gg + 1, 0, buf0, sem0)

        pltpu.make_async_copy(
            x_hbm.at[pl.ds(gg * _G, _G), pl.ds(_NP * _P, _PT)], tb,
            sem2).wait()
        pltpu.make_async_copy(
            x_hbm.at[pl.ds(gg * _G, _G), pl.ds(_NP * _P + _PT, _PE)], eb,
            sem3).wait()

        def select_blocks(r):
            iz = jnp.zeros((_L,), jnp.int32)

            def sel_body(kch, carry):
                tb1, tb2, tb3, tb4, tb5, ib1, ib2, ib3, ib4, ib5 = carry
                bm = bm_v[pl.ds((r * (_NP // 2) + kch) * _L, _L)]
                ib = lanes + kch * _L
                s1 = bm > tb1
                n1, i1 = jnp.where(s1, bm, tb1), jnp.where(s1, ib, ib1)
                bm, ib = jnp.where(s1, tb1, bm), jnp.where(s1, ib1, ib)
                s2 = bm > tb2
                n2, i2 = jnp.where(s2, bm, tb2), jnp.where(s2, ib, ib2)
                bm, ib = jnp.where(s2, tb2, bm), jnp.where(s2, ib2, ib)
                s3 = bm > tb3
                n3, i3 = jnp.where(s3, bm, tb3), jnp.where(s3, ib, ib3)
                bm, ib = jnp.where(s3, tb3, bm), jnp.where(s3, ib3, ib)
                s4 = bm > tb4
                n4, i4 = jnp.where(s4, bm, tb4), jnp.where(s4, ib, ib4)
                bm, ib = jnp.where(s4, tb4, bm), jnp.where(s4, ib4, ib)
                s5 = bm > tb5
                n5, i5 = jnp.where(s5, bm, tb5), jnp.where(s5, ib, ib5)
                return (n1, n2, n3, n4, n5, i1, i2, i3, i4, i5)

            (tb1, tb2, tb3, tb4, tb5,
             ib1, ib2, ib3, ib4, ib5) = lax.fori_loop(
                 0, _NP // 2, sel_body,
                 (ninf_vec,) * _K + (iz,) * _K)
            slots = []
            for _ in range(_K):
                m = jnp.max(tb1)
                ffs = plsc.all_reduce_ffs(tb1 == m)
                sel = lanes == ffs
                slots.append(jnp.sum(jnp.where(sel, ib1, 0)))
                tb1 = jnp.where(sel, tb2, tb1)
                tb2 = jnp.where(sel, tb3, tb2)
                tb3 = jnp.where(sel, tb4, tb3)
                tb4 = jnp.where(sel, tb5, tb4)
                tb5 = jnp.where(sel, neg_inf, tb5)
                ib1 = jnp.where(sel, ib2, ib1)
                ib2 = jnp.where(sel, ib3, ib2)
                ib3 = jnp.where(sel, ib4, ib3)
                ib4 = jnp.where(sel, ib5, ib4)
            return slots

        def slot_col(slot):
            pair = slot // _L
            lane = slot % _L
            pnl = 2 * pair + lane // _NB
            blk = lane % _NB
            return pnl * _P + blk * _BCOL

        def fetch_row_blocks(r, cols):
            for k in range(_K):
                pltpu.async_copy(
                    x_hbm.at[gg * _G + r, pl.ds(cols[k], _BCOL)],
                    rb.at[pl.ds(((r % _RBD) * _K + k) * _BCOL, _BCOL)],
                    sem4)

        def pass_b_row(r, cols, y_i):
            stack = load_stack(r)
            for k in range(_K):
                rbo = ((r % _RBD) * _K + k) * _BCOL
                def refix(rbo=rbo, col=cols[k]):
                    off = y_i - col
                    c_y = off // _L
                    l_y = off % _L
                    vy = rb[pl.ds(rbo + c_y * _L, _L)]
                    rb[pl.ds(rbo + c_y * _L, _L)] = jnp.where(
                        lanes == l_y, 0.0, vy)

                in_blk = (y_i >= cols[k]) & (y_i < cols[k] + _BCOL)
                lax.cond(in_blk, refix, lambda: None)

                def chunk_body(c, carry, rbo=rbo):
                    return _insert(carry, rb[pl.ds(rbo + c * _L, _L)])

                stack = lax.fori_loop(0, _BC, chunk_body, stack, unroll=4)
            store_stack(r, stack)

        row_cols = []
        for r in range(_G):
            slots = select_blocks(r)
            row_cols.append([slot_col(s) for s in slots])

        fetch_row_blocks(0, row_cols[0])
        fetch_row_blocks(1, row_cols[1])
        for r in range(_G):
            sy_vec = direct_span(tb, r, _NP * _P, _PT, _PTC, ys[r], sy_vec)
            sy_vec = direct_span(eb, r, _NP * _P + _PT, _PE, _PEC, ys[r],
                                 sy_vec)

        for ph in range(_G // _RB):
            r0 = ph * _RB
            for r in (r0, r0 + 1):
                for k in range(_K):
                    pltpu.make_async_copy(
                        x_hbm.at[gg * _G + r,
                                 pl.ds(row_cols[r][k], _BCOL)],
                        rb.at[pl.ds(((r % _RBD) * _K + k) * _BCOL,
                                    _BCOL)],
                        sem4).wait()
            for r in (r0, r0 + 1):
                pass_b_row(r, row_cols[r], ys[r])
            if ph + 1 < _G // _RB:
                fetch_row_blocks(r0 + 2, row_cols[r0 + 2])
                fetch_row_blocks(r0 + 3, row_cols[r0 + 3])

        for r in range(_G):
            t1, t2, t3, t4, t5 = load_stack(r)
            for _ in range(_K - 1):
                m = jnp.max(t1)
                ffs = plsc.all_reduce_ffs(t1 == m)
                sel = lanes == ffs
                t1 = jnp.where(sel, t2, t1)
                t2 = jnp.where(sel, t3, t2)
                t3 = jnp.where(sel, t4, t3)
                t4 = jnp.where(sel, t5, t4)
                t5 = jnp.where(sel, neg_inf, t5)
            s_topk = jnp.max(t1)
            s_y = jnp.sum(jnp.where(lanes == r, sy_vec, 0.0))
            hinge = jnp.maximum(1.0 + s_topk - s_y, 0.0)
            loss_acc = loss_acc + jnp.where(lane0, hinge, 0.0)
        return loss_acc

    dma_panel(wid * _NG, 0, buf0, sem0)
    loss_acc = lax.fori_loop(0, _NG, group_loop,
                             jnp.zeros((_L,), jnp.float32))

    out_v[...] = loss_acc
    pltpu.sync_copy(out_v, out_hbm.at[wid])


@jax.jit
def kernel(x, y):
    mesh = plsc.VectorSubcoreMesh(core_axis_name="c", subcore_axis_name="s")
    partials = pl.kernel(
        _body,
        out_type=jax.ShapeDtypeStruct((_NW, _L), jnp.float32),
        mesh=mesh,
        compiler_params=pltpu.CompilerParams(needs_layout_passes=False),
        scratch_types=[
            pltpu.VMEM((_G, _P), jnp.float32),
            pltpu.VMEM((_G, _P), jnp.float32),
            pltpu.VMEM((_G, _PT), jnp.float32),
            pltpu.VMEM((_G, _PE), jnp.float32),
            pltpu.VMEM((_RBD * _K * _BCOL,), jnp.float32),
            pltpu.VMEM((_RW,), jnp.int32),
            pltpu.VMEM((_G * (_NP // 2) * _L,), jnp.float32),
            pltpu.VMEM((_G * _K * _L,), jnp.float32),
            pltpu.VMEM((_L,), jnp.float32),
            pltpu.SemaphoreType.DMA,
            pltpu.SemaphoreType.DMA,
            pltpu.SemaphoreType.DMA,
            pltpu.SemaphoreType.DMA,
            pltpu.SemaphoreType.DMA,
        ],
    )(x, y)
    return jnp.sum(partials[:, 0]) / jnp.float32(_B)
